# baseline (speedup 1.0000x reference)
import jax
import jax.numpy as jnp
from jax.experimental import pallas as pl
from jax.experimental.pallas import tpu as pltpu
from jax.experimental.pallas import tpu_sc as plsc

_R = 16


def _sc_gather(x2d, rows, n, d):
    mesh = plsc.ScalarSubcoreMesh(axis_name="core", num_cores=2)
    half = n // 2

    @jax.jit
    @pl.kernel(
        out_type=jax.ShapeDtypeStruct((n, d), x2d.dtype),
        mesh=mesh,
        scratch_types=[
            pltpu.SMEM((n,), jnp.int32),
            pltpu.SemaphoreType.DMA,
            pltpu.SemaphoreType.DMA,
        ],
    )
    def gather_kernel(x_hbm, i_hbm, o_hbm, idx_s, sem_i, sem_o):
        c = jax.lax.axis_index("core")
        pltpu.async_copy(i_hbm, idx_s, sem_i).wait()
        base = c * half

        @pl.loop(0, half)
        def _(k):
            i = base + k
            pltpu.async_copy(x_hbm.at[idx_s[i]], o_hbm.at[i], sem_o)

        pltpu.make_async_copy(
            x_hbm.at[pl.ds(0, half)], o_hbm.at[pl.ds(base, half)], sem_o
        ).wait()

    return gather_kernel(x2d, rows)


def _tc_broadcast(compact, n, c, d):
    R = _R

    def body(c_ref, o_ref):
        for k in range(R):
            o_ref[k] = jnp.broadcast_to(c_ref[pl.ds(k, 1)], (c, d))

    out = pl.pallas_call(
        body,
        grid=(n // R,),
        in_specs=[pl.BlockSpec((R, d), lambda i: (i, 0))],
        out_specs=pl.BlockSpec((R, c, d), lambda i: (i, 0, 0)),
        out_shape=jax.ShapeDtypeStruct((n, c, d), compact.dtype),
    )(compact)
    return out


def kernel(logits, indices):
    N, C, H, W = logits.shape
    D = H * W
    x2d = logits.reshape(N * C, D)
    idx = indices.astype(jnp.int32)
    rows = jnp.arange(N, dtype=jnp.int32) * C + idx

    compact = _sc_gather(x2d, rows, N, D)
    out = _tc_broadcast(compact, N, C, D)
    return out.reshape(N, C, H, W)


# speedup vs baseline: 1.4872x; 1.4872x over previous
import jax
import jax.numpy as jnp
from jax.experimental import pallas as pl
from jax.experimental.pallas import tpu as pltpu

_R = 16


def kernel(logits, indices):
    N, C, H, W = logits.shape
    D = H * W
    R = _R
    x = logits.reshape(N, C, 1, D)
    idx = indices.astype(jnp.int32)

    def body(idx_ref, *refs):
        x_refs = refs[:R]
        o_ref = refs[R]
        for k in range(R):
            o_ref[k] = jnp.broadcast_to(x_refs[k][...].reshape(1, D), (C, D))

    def in_map(k):
        return lambda i, idx_ref: (i * R + k, idx_ref[i * R + k], 0, 0)

    grid_spec = pltpu.PrefetchScalarGridSpec(
        num_scalar_prefetch=1,
        grid=(N // R,),
        in_specs=[pl.BlockSpec((1, 1, 1, D), in_map(k)) for k in range(R)],
        out_specs=pl.BlockSpec((R, C, D), lambda i, idx_ref: (i, 0, 0)),
    )
    out = pl.pallas_call(
        body,
        grid_spec=grid_spec,
        out_shape=jax.ShapeDtypeStruct((N, C, D), logits.dtype),
        compiler_params=pltpu.CompilerParams(
            dimension_semantics=("parallel",),
        ),
    )(idx, *([x] * R))
    return out.reshape(N, C, H, W)
